# Initial kernel scaffold; baseline (speedup 1.0000x reference)
import jax
import jax.numpy as jnp
from jax.experimental import pallas as pl


def kernel(data_values, data_indices, idx_identity, idx_transpose, W1, b1, W2, b2, W3, b3, W4, b4):
    raise NotImplementedError("write your pallas kernel here")


# speedup vs baseline: 1.1169x; 1.1169x over previous
import functools

import jax
import jax.numpy as jnp
from jax.experimental import pallas as pl
from jax.experimental.pallas import tpu as pltpu

N_NODES = 2708
NNZ = 86656
EPS = 1e-5


def _matmul_relu_kernel(w_ref, x_ref, b_ref, o_ref, *, relu):
    z = jax.lax.dot_general(
        w_ref[...], x_ref[...], (((1,), (0,)), ((), ())),
        preferred_element_type=jnp.float32)
    z = z + b_ref[...]
    if relu:
        z = jnp.maximum(z, 0.0)
    o_ref[...] = z


def _fused_matmul(W, x, b, relu=True, block_n=2048):
    O, K = W.shape
    N = x.shape[1]
    O_pad = max(8, ((O + 7) // 8) * 8)
    if O_pad != O:
        W = jnp.pad(W, ((0, O_pad - O), (0, 0)))
        b = jnp.pad(b, (0, O_pad - O))
    grid = (N + block_n - 1) // block_n
    out = pl.pallas_call(
        functools.partial(_matmul_relu_kernel, relu=relu),
        grid=(grid,),
        in_specs=[
            pl.BlockSpec((O_pad, K), lambda i: (0, 0)),
            pl.BlockSpec((K, block_n), lambda i: (0, i)),
            pl.BlockSpec((O_pad, 1), lambda i: (0, 0)),
        ],
        out_specs=pl.BlockSpec((O_pad, block_n), lambda i: (0, i)),
        out_shape=jax.ShapeDtypeStruct((O_pad, N), jnp.float32),
    )(W, x, b[:, None])
    return out[:O]


def _pool(v, seg):
    return jax.ops.segment_sum(v.T, seg, num_segments=N_NODES).T


def _norm(x):
    mean = x.mean(axis=1, keepdims=True)
    std = x.std(axis=1, keepdims=True)
    return (x - mean) / (std + EPS)


def _layer(v, row, col, idx_id, idx_tr, W, b):
    C = v.shape[0]
    rowsum = _pool(v, row)
    colsum = _pool(v, col)
    g = v.sum(axis=1, keepdims=True)
    ops = jnp.concatenate([
        v, v[:, idx_tr], rowsum[:, row], colsum[:, col],
        jnp.broadcast_to(g, v.shape), v[:, idx_id],
    ], axis=0)
    Wf = jnp.transpose(W, (1, 0, 2)).reshape(W.shape[1], 6 * C)
    return _fused_matmul(Wf, ops, b, relu=True)


def kernel(data_values, data_indices, idx_identity, idx_transpose,
           W1, b1, W2, b2, W3, b3, W4, b4):
    row = data_indices[0]
    col = data_indices[1]
    v = data_values
    out = _norm(_layer(v, row, col, idx_identity, idx_transpose, W1, b1))
    out = _norm(_layer(out, row, col, idx_identity, idx_transpose, W2, b2))
    out = _norm(_layer(out, row, col, idx_identity, idx_transpose, W3, b3))
    rowsum = _pool(out, row)
    colsum = _pool(out, col)
    g = jnp.broadcast_to(out.sum(axis=1, keepdims=True), rowsum.shape)
    ops4 = jnp.concatenate([rowsum, colsum, g], axis=0)
    W4f = jnp.transpose(W4, (1, 0, 2)).reshape(W4.shape[1], 3 * W4.shape[2])
    out = _fused_matmul(W4f, ops4, b4, relu=False)
    data_out = out.reshape(out.shape[0], N_NODES).T
    return jax.nn.softmax(data_out, axis=1)
